# baseline (speedup 1.0000x reference)
import functools

import jax
import jax.numpy as jnp
from jax import lax
from jax.experimental import pallas as pl
from jax.experimental.pallas import tpu as pltpu
from jax.experimental.pallas import tpu_sc as plsc

B = 16384
ND = 13
NS = 26
E = 16
V = 100000
D0 = (ND + NS) * E
H1 = 512
H2 = 256
DE = ND * E
SE = NS * E

NC = 2
NSUB = 16
NW = NC * NSUB
ROWS = B * NS
RPW = ROWS // NW
CH = 1664
NCHUNK = RPW // CH
SUBG = 128
NSUBG = CH // SUBG


def _sc_gather_body(t1, t2, idxh, w1h, s1p, p2, idx_v, rows_v, w1_v,
                    sacc_v, sem):
    c = lax.axis_index("c")
    s = lax.axis_index("s")
    wid = s * NC + c
    base_w = wid * RPW
    lane = lax.iota(jnp.int32, 16)

    def chunk(k, carry):
        base = pl.multiple_of(base_w + k * CH, 8)
        pltpu.sync_copy(idxh.at[pl.ds(base, CH)], idx_v)

        @plsc.parallel_loop(0, CH // 16, unroll=4)
        def _fix(i):
            pos = base + i * 16 + lane
            v = idx_v[pl.ds(i * 16, 16)]
            idx_v[pl.ds(i * 16, 16)] = v + (pos % NS) * V
        pltpu.sync_copy(w1h.at[pl.ds(base, CH)], w1_v)

        cps = [
            pltpu.async_copy(
                t1.at[idx_v.at[pl.ds(g * SUBG, SUBG)]],
                rows_v.at[pl.ds(g * SUBG, SUBG)],
                sem,
            )
            for g in range(NSUBG)
        ]
        for cp in cps:
            cp.wait()

        @plsc.parallel_loop(0, CH // NS, unroll=2)
        def _bag(bl):
            r0 = bl * NS
            acc = rows_v[r0, :] * w1_v[r0, :]
            for s2 in range(1, NS):
                acc = acc + rows_v[r0 + s2, :] * w1_v[r0 + s2, :]
            sacc_v[bl, :] = acc
        pltpu.sync_copy(sacc_v, s1p.at[pl.ds(base // NS, CH // NS)])

        cps = [
            pltpu.async_copy(
                t2.at[idx_v.at[pl.ds(g * SUBG, SUBG)]],
                rows_v.at[pl.ds(g * SUBG, SUBG)],
                sem,
            )
            for g in range(NSUBG)
        ]
        for cp in cps:
            cp.wait()

        @plsc.parallel_loop(0, CH // 8, unroll=2)
        def _mul(i):
            r0 = i * 8
            for o in range(8):
                rows_v[r0 + o, :] = rows_v[r0 + o, :] * w1_v[r0 + o, :]
        pltpu.sync_copy(rows_v, p2.at[pl.ds(base, CH)])
        return carry

    lax.fori_loop(0, NCHUNK, chunk, 0)


@functools.cache
def _get_sc_gather():
    return functools.partial(
        pl.kernel,
        out_type=[
            jax.ShapeDtypeStruct((B, E), jnp.float32),
            jax.ShapeDtypeStruct((ROWS, E), jnp.float32),
        ],
        mesh=plsc.VectorSubcoreMesh(
            core_axis_name="c", subcore_axis_name="s",
            num_cores=NC, num_subcores=NSUB,
        ),
        scratch_types=[
            pltpu.VMEM((CH,), jnp.int32),
            pltpu.VMEM((CH, E), jnp.float32),
            pltpu.VMEM((CH, E), jnp.float32),
            pltpu.VMEM((CH // NS, E), jnp.float32),
            pltpu.SemaphoreType.DMA,
        ],
        compiler_params=pltpu.CompilerParams(use_tc_tiling_on_sc=False),
    )(_sc_gather_body)


BK = 1024
G = B // BK


def _expand_mat(wf):
    d = lax.broadcasted_iota(jnp.int32, (ND, DE), 0)
    j = lax.broadcasted_iota(jnp.int32, (ND, DE), 1)
    return jnp.where(j // E == d, jnp.broadcast_to(wf, (ND, DE)), 0.0)


def _k1_body(xi, w1f, b1f, co1):
    w1e = _expand_mat(w1f[0:1, :])
    co1[...] = (jnp.dot(xi[...], w1e, preferred_element_type=jnp.float32, precision=lax.Precision.HIGHEST)
                + b1f[0:1, :])


def _make_k1(interpret=False):
    return pl.pallas_call(
        _k1_body,
        grid=(G,),
        in_specs=[
            pl.BlockSpec((BK, ND), lambda i: (i, 0)),
            pl.BlockSpec((8, DE), lambda i: (0, 0)),
            pl.BlockSpec((8, DE), lambda i: (0, 0)),
        ],
        out_specs=pl.BlockSpec((BK, DE), lambda i: (i, 0)),
        out_shape=jax.ShapeDtypeStruct((B, DE), jnp.float32),
        interpret=interpret,
    )


def _p1_body(s1p, p2, y, xvd, xi, w2f, b2f, l1w, l1b,
             sfm, h1, hsum, hsq):
    i = pl.program_id(0)
    f32 = jnp.float32
    md = (lax.broadcasted_iota(jnp.int32, (DE, ND), 0) // E
          == lax.broadcasted_iota(jnp.int32, (DE, ND), 1)).astype(f32)
    ys = jnp.dot(y[...], md, preferred_element_type=f32, precision=lax.Precision.HIGHEST)
    s1 = jnp.sum(s1p[...], axis=1) + jnp.sum(ys * xvd[...], axis=1)
    w2e = _expand_mat(w2f[0:1, :])
    co2 = jnp.dot(xi[...], w2e, preferred_element_type=f32, precision=lax.Precision.HIGHEST) + b2f[0:1, :]
    P2 = p2[...]
    r208 = lax.broadcasted_iota(jnp.int32, (DE, E), 0) % E
    m208 = (r208 == lax.broadcasted_iota(jnp.int32, (DE, E), 1)).astype(jnp.float32)
    r416 = lax.broadcasted_iota(jnp.int32, (SE, E), 0) % E
    m416 = (r416 == lax.broadcasted_iota(jnp.int32, (SE, E), 1)).astype(jnp.float32)
    s = (jnp.dot(co2, m208, preferred_element_type=f32, precision=lax.Precision.HIGHEST)
         + jnp.dot(P2, m416, preferred_element_type=f32, precision=lax.Precision.HIGHEST))
    sqs = (jnp.dot(co2 * co2, m208, preferred_element_type=f32, precision=lax.Precision.HIGHEST)
           + jnp.dot(P2 * P2, m416, preferred_element_type=f32, precision=lax.Precision.HIGHEST))
    S2 = 0.5 * jnp.sum(s * s - sqs, axis=1)
    sfm[...] = s1 + S2
    h = (jnp.dot(co2, l1w[0:DE, :], preferred_element_type=f32)
         + jnp.dot(P2, l1w[DE:D0, :], preferred_element_type=f32)
         + l1b[...])
    h1[...] = h

    @pl.when(i == 0)
    def _():
        hsum[...] = jnp.zeros_like(hsum)
        hsq[...] = jnp.zeros_like(hsq)

    hsum[...] += jnp.sum(h, axis=0)
    hsq[...] += jnp.sum(h * h, axis=0)


def _make_p1(interpret=False):
    full512 = pl.BlockSpec((H1,), lambda i: (0,))
    return pl.pallas_call(
        _p1_body,
        grid=(G,),
        in_specs=[
            pl.BlockSpec((BK, E), lambda i: (i, 0)),
            pl.BlockSpec((BK, SE), lambda i: (i, 0)),
            pl.BlockSpec((BK, DE), lambda i: (i, 0)),
            pl.BlockSpec((BK, ND), lambda i: (i, 0)),
            pl.BlockSpec((BK, ND), lambda i: (i, 0)),
            pl.BlockSpec((8, DE), lambda i: (0, 0)),
            pl.BlockSpec((8, DE), lambda i: (0, 0)),
            pl.BlockSpec((D0, H1), lambda i: (0, 0)),
            full512,
        ],
        out_specs=[
            pl.BlockSpec((BK,), lambda i: (i,)),
            pl.BlockSpec((BK, H1), lambda i: (i, 0)),
            full512,
            full512,
        ],
        out_shape=[
            jax.ShapeDtypeStruct((B,), jnp.float32),
            jax.ShapeDtypeStruct((B, H1), jnp.float32),
            jax.ShapeDtypeStruct((H1,), jnp.float32),
            jax.ShapeDtypeStruct((H1,), jnp.float32),
        ],
        interpret=interpret,
    )


def _p2_body(h1, hsum, hsq, g1, l2w, u, usum, usq):
    i = pl.program_id(0)
    mu = hsum[...] * (1.0 / B)
    va = hsq[...] * (1.0 / B) - mu * mu
    a1 = g1[...] * lax.rsqrt(va + 1e-5)
    zn = (h1[...] - mu) * a1
    uu = jnp.dot(zn, l2w[...], preferred_element_type=jnp.float32)
    u[...] = uu

    @pl.when(i == 0)
    def _():
        usum[...] = jnp.zeros_like(usum)
        usq[...] = jnp.zeros_like(usq)

    usum[...] += jnp.sum(uu, axis=0)
    usq[...] += jnp.sum(uu * uu, axis=0)


def _make_p2(interpret=False):
    full512 = pl.BlockSpec((H1,), lambda i: (0,))
    full256 = pl.BlockSpec((H2,), lambda i: (0,))
    return pl.pallas_call(
        _p2_body,
        grid=(G,),
        in_specs=[
            pl.BlockSpec((BK, H1), lambda i: (i, 0)),
            full512,
            full512,
            full512,
            pl.BlockSpec((H1, H2), lambda i: (0, 0)),
        ],
        out_specs=[
            pl.BlockSpec((BK, H2), lambda i: (i, 0)),
            full256,
            full256,
        ],
        out_shape=[
            jax.ShapeDtypeStruct((B, H2), jnp.float32),
            jax.ShapeDtypeStruct((H2,), jnp.float32),
            jax.ShapeDtypeStruct((H2,), jnp.float32),
        ],
        interpret=interpret,
    )


def _p3_body(u, usum, usq, g2, bb2, sfm, bias, out):
    mu = usum[...] * (1.0 / B)
    va = usq[...] * (1.0 / B) - mu * mu
    a2 = g2[...] * lax.rsqrt(va + 1e-5)
    c3 = jnp.sum(bb2[...]) - jnp.sum(mu * a2)
    s3 = jnp.sum(u[...] * a2, axis=1) + c3
    out[...] = sfm[...] + s3 + bias[...]


def _make_p3(interpret=False):
    full256 = pl.BlockSpec((H2,), lambda i: (0,))
    vec = pl.BlockSpec((BK,), lambda i: (i,))
    return pl.pallas_call(
        _p3_body,
        grid=(G,),
        in_specs=[
            pl.BlockSpec((BK, H2), lambda i: (i, 0)),
            full256,
            full256,
            full256,
            full256,
            vec,
            vec,
        ],
        out_specs=vec,
        out_shape=jax.ShapeDtypeStruct((B,), jnp.float32),
        interpret=interpret,
    )


_k1 = _make_k1()
_p1 = _make_p1()
_p2 = _make_p2()
_p3 = _make_p3()


@jax.jit
def kernel(Xi, Xv, conv1_W, conv1_b, conv2_W, conv2_b, emb1, emb2,
           lin1_W, lin1_b, bn1_g, bn1_b, lin2_W, lin2_b, bn2_g, bn2_b, bias):
    Xi_lin = Xi[:, :ND, 0].astype(jnp.float32)
    idx_flat = Xi[:, ND:, 0].reshape(-1)
    W1 = jnp.broadcast_to(Xv[:, ND:].reshape(NS, 1, B), (NS, E, B)).reshape(B, SE)
    XvD = Xv[:, :ND]
    w1f = jnp.broadcast_to(conv1_W.reshape(1, DE), (8, DE))
    b1f = jnp.broadcast_to(conv1_b.reshape(1, DE), (8, DE))
    w2f = jnp.broadcast_to(conv2_W.reshape(1, DE), (8, DE))
    b2f = jnp.broadcast_to(conv2_b.reshape(1, DE), (8, DE))

    s1p, P2f = _get_sc_gather()(emb1.reshape(NS * V, E),
                                emb2.reshape(NS * V, E), idx_flat,
                                W1.reshape(ROWS, E))
    P2 = P2f.reshape(B, SE)

    co1 = _k1(Xi_lin, w1f, b1f)
    Y = co1.T.reshape(B, DE)

    sfm, h1, hsum, hsq = _p1(s1p, P2, Y, XvD, Xi_lin, w2f, b2f,
                             lin1_W, lin1_b)
    u, usum, usq = _p2(h1, hsum, hsq, bn1_g, lin2_W)
    return _p3(u, usum, usq, bn2_g, bn2_b, sfm, bias)


# speedup vs baseline: 1.0223x; 1.0223x over previous
import functools

import jax
import jax.numpy as jnp
from jax import lax
from jax.experimental import pallas as pl
from jax.experimental.pallas import tpu as pltpu
from jax.experimental.pallas import tpu_sc as plsc

B = 16384
ND = 13
NS = 26
E = 16
V = 100000
D0 = (ND + NS) * E
H1 = 512
H2 = 256
DE = ND * E
SE = NS * E

NC = 2
NSUB = 16
NW = NC * NSUB
ROWS = B * NS
RPW = ROWS // NW
CH = 1664
NCHUNK = RPW // CH
SUBG = 128
NSUBG = CH // SUBG


def _fix_indices(idx_v, base, lane):
    @plsc.parallel_loop(0, CH // 16, unroll=4)
    def _fix(i):
        pos = base + i * 16 + lane
        v = idx_v[pl.ds(i * 16, 16)]
        idx_v[pl.ds(i * 16, 16)] = v + (pos % NS) * V


def _gather_chunk(tab, idx_v, rows_v, sem):
    cps = [
        pltpu.async_copy(
            tab.at[idx_v.at[pl.ds(g * SUBG, SUBG)]],
            rows_v.at[pl.ds(g * SUBG, SUBG)],
            sem,
        )
        for g in range(NSUBG)
    ]
    for cp in cps:
        cp.wait()


def _sc_first_body(t1, idxh, w1h, s1p, idx_v, rows_v, w1_v, sacc_v, sem):
    c = lax.axis_index("c")
    s = lax.axis_index("s")
    base_w = (s * NC + c) * RPW
    lane = lax.iota(jnp.int32, 16)

    def chunk(k, carry):
        base = pl.multiple_of(base_w + k * CH, 8)
        pltpu.sync_copy(idxh.at[pl.ds(base, CH)], idx_v)
        _fix_indices(idx_v, base, lane)
        pltpu.sync_copy(w1h.at[pl.ds(base, CH)], w1_v)
        _gather_chunk(t1, idx_v, rows_v, sem)

        @plsc.parallel_loop(0, CH // NS, unroll=2)
        def _bag(bl):
            r0 = bl * NS
            acc = rows_v[r0, :] * w1_v[r0, :]
            for s2 in range(1, NS):
                acc = acc + rows_v[r0 + s2, :] * w1_v[r0 + s2, :]
            sacc_v[bl, :] = acc
        pltpu.sync_copy(sacc_v, s1p.at[pl.ds(base // NS, CH // NS)])
        return carry

    lax.fori_loop(0, NCHUNK, chunk, 0)


def _sc_second_body(t2, idxh, w1h, p2, idx_v, rows_v, w1_v, sem):
    c = lax.axis_index("c")
    s = lax.axis_index("s")
    base_w = (s * NC + c) * RPW
    lane = lax.iota(jnp.int32, 16)

    def chunk(k, carry):
        base = pl.multiple_of(base_w + k * CH, 8)
        pltpu.sync_copy(idxh.at[pl.ds(base, CH)], idx_v)
        _fix_indices(idx_v, base, lane)
        pltpu.sync_copy(w1h.at[pl.ds(base, CH)], w1_v)
        _gather_chunk(t2, idx_v, rows_v, sem)

        @plsc.parallel_loop(0, CH // 8, unroll=2)
        def _mul(i):
            r0 = i * 8
            for o in range(8):
                rows_v[r0 + o, :] = rows_v[r0 + o, :] * w1_v[r0 + o, :]
        pltpu.sync_copy(rows_v, p2.at[pl.ds(base, CH)])
        return carry

    lax.fori_loop(0, NCHUNK, chunk, 0)


def _sc_mesh():
    return plsc.VectorSubcoreMesh(
        core_axis_name="c", subcore_axis_name="s",
        num_cores=NC, num_subcores=NSUB,
    )


@functools.cache
def _get_sc_first():
    return functools.partial(
        pl.kernel,
        out_type=jax.ShapeDtypeStruct((B, E), jnp.float32),
        mesh=_sc_mesh(),
        scratch_types=[
            pltpu.VMEM((CH,), jnp.int32),
            pltpu.VMEM((CH, E), jnp.float32),
            pltpu.VMEM((CH, E), jnp.float32),
            pltpu.VMEM((CH // NS, E), jnp.float32),
            pltpu.SemaphoreType.DMA,
        ],
        compiler_params=pltpu.CompilerParams(use_tc_tiling_on_sc=False),
    )(_sc_first_body)


@functools.cache
def _get_sc_second():
    return functools.partial(
        pl.kernel,
        out_type=jax.ShapeDtypeStruct((ROWS, E), jnp.float32),
        mesh=_sc_mesh(),
        scratch_types=[
            pltpu.VMEM((CH,), jnp.int32),
            pltpu.VMEM((CH, E), jnp.float32),
            pltpu.VMEM((CH, E), jnp.float32),
            pltpu.SemaphoreType.DMA,
        ],
        compiler_params=pltpu.CompilerParams(use_tc_tiling_on_sc=False),
    )(_sc_second_body)


BK = 1024
G = B // BK


def _expand_mat(wf):
    d = lax.broadcasted_iota(jnp.int32, (ND, DE), 0)
    j = lax.broadcasted_iota(jnp.int32, (ND, DE), 1)
    return jnp.where(j // E == d, jnp.broadcast_to(wf, (ND, DE)), 0.0)


def _k1_body(xi, w1f, b1f, co1):
    w1e = _expand_mat(w1f[0:1, :])
    co1[...] = (jnp.dot(xi[...], w1e, preferred_element_type=jnp.float32, precision=lax.Precision.HIGHEST)
                + b1f[0:1, :])


def _make_k1(interpret=False):
    return pl.pallas_call(
        _k1_body,
        grid=(G,),
        in_specs=[
            pl.BlockSpec((BK, ND), lambda i: (i, 0)),
            pl.BlockSpec((8, DE), lambda i: (0, 0)),
            pl.BlockSpec((8, DE), lambda i: (0, 0)),
        ],
        out_specs=pl.BlockSpec((BK, DE), lambda i: (i, 0)),
        out_shape=jax.ShapeDtypeStruct((B, DE), jnp.float32),
        interpret=interpret,
    )


def _p1_body(p2, y, xvd, xi, w2f, b2f, l1w, l1b,
             sfm, h1, hsum, hsq):
    i = pl.program_id(0)
    f32 = jnp.float32
    md = (lax.broadcasted_iota(jnp.int32, (DE, ND), 0) // E
          == lax.broadcasted_iota(jnp.int32, (DE, ND), 1)).astype(f32)
    ys = jnp.dot(y[...], md, preferred_element_type=f32, precision=lax.Precision.HIGHEST)
    s1 = jnp.sum(ys * xvd[...], axis=1)
    w2e = _expand_mat(w2f[0:1, :])
    co2 = jnp.dot(xi[...], w2e, preferred_element_type=f32, precision=lax.Precision.HIGHEST) + b2f[0:1, :]
    P2 = p2[...]
    r208 = lax.broadcasted_iota(jnp.int32, (DE, E), 0) % E
    m208 = (r208 == lax.broadcasted_iota(jnp.int32, (DE, E), 1)).astype(jnp.float32)
    r416 = lax.broadcasted_iota(jnp.int32, (SE, E), 0) % E
    m416 = (r416 == lax.broadcasted_iota(jnp.int32, (SE, E), 1)).astype(jnp.float32)
    s = (jnp.dot(co2, m208, preferred_element_type=f32, precision=lax.Precision.HIGHEST)
         + jnp.dot(P2, m416, preferred_element_type=f32, precision=lax.Precision.HIGHEST))
    sqs = (jnp.dot(co2 * co2, m208, preferred_element_type=f32, precision=lax.Precision.HIGHEST)
           + jnp.dot(P2 * P2, m416, preferred_element_type=f32, precision=lax.Precision.HIGHEST))
    S2 = 0.5 * jnp.sum(s * s - sqs, axis=1)
    sfm[...] = s1 + S2
    h = (jnp.dot(co2, l1w[0:DE, :], preferred_element_type=f32)
         + jnp.dot(P2, l1w[DE:D0, :], preferred_element_type=f32)
         + l1b[...])
    h1[...] = h

    @pl.when(i == 0)
    def _():
        hsum[...] = jnp.zeros_like(hsum)
        hsq[...] = jnp.zeros_like(hsq)

    hsum[...] += jnp.sum(h, axis=0)
    hsq[...] += jnp.sum(h * h, axis=0)


def _make_p1(interpret=False):
    full512 = pl.BlockSpec((H1,), lambda i: (0,))
    return pl.pallas_call(
        _p1_body,
        grid=(G,),
        in_specs=[
            pl.BlockSpec((BK, SE), lambda i: (i, 0)),
            pl.BlockSpec((BK, DE), lambda i: (i, 0)),
            pl.BlockSpec((BK, ND), lambda i: (i, 0)),
            pl.BlockSpec((BK, ND), lambda i: (i, 0)),
            pl.BlockSpec((8, DE), lambda i: (0, 0)),
            pl.BlockSpec((8, DE), lambda i: (0, 0)),
            pl.BlockSpec((D0, H1), lambda i: (0, 0)),
            full512,
        ],
        out_specs=[
            pl.BlockSpec((BK,), lambda i: (i,)),
            pl.BlockSpec((BK, H1), lambda i: (i, 0)),
            full512,
            full512,
        ],
        out_shape=[
            jax.ShapeDtypeStruct((B,), jnp.float32),
            jax.ShapeDtypeStruct((B, H1), jnp.float32),
            jax.ShapeDtypeStruct((H1,), jnp.float32),
            jax.ShapeDtypeStruct((H1,), jnp.float32),
        ],
        interpret=interpret,
    )


def _p2_body(h1, hsum, hsq, g1, l2w, u, usum, usq):
    i = pl.program_id(0)
    mu = hsum[...] * (1.0 / B)
    va = hsq[...] * (1.0 / B) - mu * mu
    a1 = g1[...] * lax.rsqrt(va + 1e-5)
    zn = (h1[...] - mu) * a1
    uu = jnp.dot(zn, l2w[...], preferred_element_type=jnp.float32)
    u[...] = uu

    @pl.when(i == 0)
    def _():
        usum[...] = jnp.zeros_like(usum)
        usq[...] = jnp.zeros_like(usq)

    usum[...] += jnp.sum(uu, axis=0)
    usq[...] += jnp.sum(uu * uu, axis=0)


def _make_p2(interpret=False):
    full512 = pl.BlockSpec((H1,), lambda i: (0,))
    full256 = pl.BlockSpec((H2,), lambda i: (0,))
    return pl.pallas_call(
        _p2_body,
        grid=(G,),
        in_specs=[
            pl.BlockSpec((BK, H1), lambda i: (i, 0)),
            full512,
            full512,
            full512,
            pl.BlockSpec((H1, H2), lambda i: (0, 0)),
        ],
        out_specs=[
            pl.BlockSpec((BK, H2), lambda i: (i, 0)),
            full256,
            full256,
        ],
        out_shape=[
            jax.ShapeDtypeStruct((B, H2), jnp.float32),
            jax.ShapeDtypeStruct((H2,), jnp.float32),
            jax.ShapeDtypeStruct((H2,), jnp.float32),
        ],
        interpret=interpret,
    )


def _p3_body(u, usum, usq, g2, bb2, sfm, s1p, bias, out):
    mu = usum[...] * (1.0 / B)
    va = usq[...] * (1.0 / B) - mu * mu
    a2 = g2[...] * lax.rsqrt(va + 1e-5)
    c3 = jnp.sum(bb2[...]) - jnp.sum(mu * a2)
    s3 = jnp.sum(u[...] * a2, axis=1) + c3
    out[...] = sfm[...] + jnp.sum(s1p[...], axis=1) + s3 + bias[...]


def _make_p3(interpret=False):
    full256 = pl.BlockSpec((H2,), lambda i: (0,))
    vec = pl.BlockSpec((BK,), lambda i: (i,))
    return pl.pallas_call(
        _p3_body,
        grid=(G,),
        in_specs=[
            pl.BlockSpec((BK, H2), lambda i: (i, 0)),
            full256,
            full256,
            full256,
            full256,
            vec,
            pl.BlockSpec((BK, E), lambda i: (i, 0)),
            vec,
        ],
        out_specs=vec,
        out_shape=jax.ShapeDtypeStruct((B,), jnp.float32),
        interpret=interpret,
    )


_k1 = _make_k1()
_p1 = _make_p1()
_p2 = _make_p2()
_p3 = _make_p3()


@jax.jit
def kernel(Xi, Xv, conv1_W, conv1_b, conv2_W, conv2_b, emb1, emb2,
           lin1_W, lin1_b, bn1_g, bn1_b, lin2_W, lin2_b, bn2_g, bn2_b, bias):
    Xi_lin = Xi[:, :ND, 0].astype(jnp.float32)
    idx_flat = Xi[:, ND:, 0].reshape(-1)
    W1 = jnp.broadcast_to(Xv[:, ND:].reshape(NS, 1, B), (NS, E, B)).reshape(B, SE)
    XvD = Xv[:, :ND]
    w1f = jnp.broadcast_to(conv1_W.reshape(1, DE), (8, DE))
    b1f = jnp.broadcast_to(conv1_b.reshape(1, DE), (8, DE))
    w2f = jnp.broadcast_to(conv2_W.reshape(1, DE), (8, DE))
    b2f = jnp.broadcast_to(conv2_b.reshape(1, DE), (8, DE))

    w1flat = W1.reshape(ROWS, E)
    P2f = _get_sc_second()(emb2.reshape(NS * V, E), idx_flat, w1flat)
    s1p = _get_sc_first()(emb1.reshape(NS * V, E), idx_flat, w1flat)
    P2 = P2f.reshape(B, SE)

    co1 = _k1(Xi_lin, w1f, b1f)
    Y = co1.T.reshape(B, DE)

    sfm, h1, hsum, hsq = _p1(P2, Y, XvD, Xi_lin, w2f, b2f,
                             lin1_W, lin1_b)
    u, usum, usq = _p2(h1, hsum, hsq, bn1_g, lin2_W)
    return _p3(u, usum, usq, bn2_g, bn2_b, sfm, s1p, bias)
